# baseline (speedup 1.0000x reference)
import jax
import jax.numpy as jnp
from itertools import product as _prod
from jax.experimental import pallas as pl
from jax.experimental.pallas import tpu as pltpu

_VSZ = 0.1
_FD = 128
_NBLK = 2
_NTILE = 2000


def _structure(points):
    N = points.shape[0]
    coords = jnp.floor(points / _VSZ).astype(jnp.int64)
    coords = coords - coords.min(axis=0)
    mx = coords.max(axis=0) + 1
    mx1, mx2 = mx[1], mx[2]
    h = coords[:, 0] * (mx1 * mx2) + coords[:, 1] * mx2 + coords[:, 2]
    uh, inv = jnp.unique(h, return_inverse=True, size=N, fill_value=-1)
    inv = inv.reshape(-1).astype(jnp.int32)
    M = jnp.sum(uh >= 0).astype(jnp.int32)
    row_valid = jnp.arange(N, dtype=jnp.int32) < M
    c0 = uh // (mx1 * mx2)
    r = uh % (mx1 * mx2)
    c1 = r // mx2
    c2 = r % mx2
    vc = jnp.stack([c0, c1, c2], axis=1)
    big = jnp.iinfo(uh.dtype).max
    uh_s = jnp.where(row_valid, uh, big)
    offs = jnp.array(list(_prod((-1, 0, 1), repeat=3)), dtype=vc.dtype)
    nc = vc[None, :, :] + offs[:, None, :]
    valid = (jnp.all((nc >= 0) & (nc < mx[None, None, :]), axis=2)
             & row_valid[None, :])
    nh = nc[..., 0] * (mx1 * mx2) + nc[..., 1] * mx2 + nc[..., 2]
    pos = jnp.searchsorted(uh_s, nh.reshape(-1)).astype(jnp.int32)
    pos_c = jnp.clip(pos, 0, N - 1).reshape(27, N)
    hit = uh_s[pos_c] == nh
    neigh = jnp.where(valid & hit, pos_c, -1)
    return inv, M, neigh


def _conv_mm(G, W):
    N = G.shape[1]
    nt = N // _NTILE

    def body(g_ref, w_ref, o_ref):
        k = pl.program_id(1)

        @pl.when(k == 0)
        def _():
            o_ref[...] = jnp.zeros_like(o_ref)

        o_ref[...] += jnp.dot(g_ref[0], w_ref[0],
                              preferred_element_type=jnp.float32)

    return pl.pallas_call(
        body,
        grid=(nt, 27),
        in_specs=[
            pl.BlockSpec((1, _NTILE, _FD), lambda i, k: (k, i, 0)),
            pl.BlockSpec((1, _FD, _FD), lambda i, k: (k, 0, 0)),
        ],
        out_specs=pl.BlockSpec((_NTILE, _FD), lambda i, k: (i, 0)),
        out_shape=jax.ShapeDtypeStruct((N, _FD), jnp.float32),
        compiler_params=pltpu.CompilerParams(
            dimension_semantics=("parallel", "arbitrary")),
    )(G, W)


def kernel(source_points, target_points, enc_W, enc_b, ln_g, ln_b, convW,
           bn_g, bn_b, h1_W, h1_b, h2_W, h2_b):
    sc = source_points - source_points.mean(axis=0, keepdims=True)
    tc = target_points - target_points.mean(axis=0, keepdims=True)
    s_inv, s_M, s_nb = _structure(sc)
    t_inv, t_M, t_nb = _structure(tc)
    Npad = source_points.shape[0]

    def encode(p):
        x = p @ enc_W + enc_b
        m = x.mean(axis=-1, keepdims=True)
        v = ((x - m) ** 2).mean(axis=-1, keepdims=True)
        x = (x - m) / jnp.sqrt(v + 1e-5) * ln_g + ln_b
        return jax.nn.relu(x)

    def vox_mean(feats, inv):
        s = jax.ops.segment_sum(feats, inv, num_segments=Npad)
        c = jax.ops.segment_sum(jnp.ones((feats.shape[0],), feats.dtype),
                                inv, num_segments=Npad)
        return s / jnp.where(c > 0, c, jnp.ones_like(c))[:, None]

    def subm(f, W, nb):
        G = jnp.where(nb[:, :, None] >= 0, f[jnp.clip(nb, 0)], 0.0)
        return _conv_mm(G, W)

    def bn(x, g, b, mask, Mf):
        m = jnp.where(mask[:, None], x, 0.0).sum(axis=0) / Mf
        v = jnp.where(mask[:, None], (x - m) ** 2, 0.0).sum(axis=0) / Mf
        return (x - m) / jnp.sqrt(v + 1e-5) * g + b

    def blocks(f, nb, mask, Mf):
        x = f
        for bi in range(_NBLK):
            idn = x
            y = jax.nn.relu(bn(subm(x, convW[bi, 0], nb),
                               bn_g[bi, 0], bn_b[bi, 0], mask, Mf))
            y = bn(subm(y, convW[bi, 1], nb),
                   bn_g[bi, 1], bn_b[bi, 1], mask, Mf) + idn
            x = jax.nn.relu(y)
        return x

    s_mask = jnp.arange(Npad, dtype=jnp.int32) < s_M
    t_mask = jnp.arange(Npad, dtype=jnp.int32) < t_M
    s_Mf = s_M.astype(jnp.float32)
    t_Mf = t_M.astype(jnp.float32)

    s_feats = blocks(vox_mean(encode(source_points), s_inv), s_nb, s_mask, s_Mf)
    t_feats = blocks(vox_mean(encode(target_points), t_inv), t_nb, t_mask, t_Mf)
    sg = jnp.where(s_mask[:, None], s_feats, -jnp.inf).max(axis=0)
    tg = jnp.where(t_mask[:, None], t_feats, -jnp.inf).max(axis=0)
    comb = sg + tg
    h = jax.nn.relu(comb @ h1_W + h1_b)
    return h @ h2_W + h2_b


# speedup vs baseline: 7.9119x; 7.9119x over previous
import functools

import jax
import jax.numpy as jnp
from jax import lax
from itertools import product as _prod
from jax.experimental import pallas as pl
from jax.experimental.pallas import tpu as pltpu
from jax.experimental.pallas import tpu_sc as plsc

_VSZ = 0.1
_FD = 128
_NBLK = 2
_NTILE = 2000

_T = 2 ** 23
_TBLK = 16384
_INIT_BLKS = _T // _TBLK + 1
_TALLOC = _INIT_BLKS * _TBLK
_DUMP = _T
_MISS = _T + 64
_NSROW = 512
_NGROW = 11264


def _sc_table_lookup(sidx2, sval2, gidx2):
    mesh = plsc.VectorSubcoreMesh(core_axis_name="c", subcore_axis_name="s",
                                  num_cores=1)

    @functools.partial(
        pl.kernel,
        out_type=(jax.ShapeDtypeStruct((_NGROW, 128), jnp.int32),
                  jax.ShapeDtypeStruct((_TALLOC,), jnp.int32)),
        mesh=mesh,
        scratch_types=[
            pltpu.VMEM((_TBLK,), jnp.int32),
            pltpu.VMEM((32, 128), jnp.int32),
            pltpu.VMEM((32, 128), jnp.int32),
            pltpu.VMEM((88, 128), jnp.int32),
            pltpu.VMEM((88, 128), jnp.int32),
            pltpu.SemaphoreType.DMA,
        ],
    )
    def k(sidx_ref, sval_ref, gidx_ref, out_ref, table_ref,
          neg_v, sidx_v, sval_v, gix_v, res_v, sem):
        t = lax.axis_index("s")
        m1 = jnp.full((16,), -1, jnp.int32)

        @pl.loop(0, _TBLK // 16)
        def _(i):
            neg_v[pl.ds(i * 16, 16)] = m1

        @pl.loop(0, 33)
        def _(j):
            blk = j * 16 + t

            @pl.when(blk < _INIT_BLKS)
            def _():
                pltpu.sync_copy(neg_v, table_ref.at[pl.ds(blk * _TBLK, _TBLK)])

        plsc.subcore_barrier()

        pltpu.sync_copy(sidx_ref.at[pl.ds(t * 32, 32), :], sidx_v)
        pltpu.sync_copy(sval_ref.at[pl.ds(t * 32, 32), :], sval_v)

        @pl.loop(0, 32)
        def _(c):
            pltpu.async_copy(sval_v.at[c], table_ref.at[sidx_v.at[c]],
                             sem).wait()

        plsc.subcore_barrier()

        @pl.loop(0, 8)
        def _(b):
            row0 = t * (_NGROW // 16) + b * 88
            pltpu.sync_copy(gidx_ref.at[pl.ds(row0, 88), :], gix_v)

            @pl.loop(0, 11)
            def _(j):
                descs = [pltpu.async_copy(table_ref.at[gix_v.at[j * 8 + i]],
                                          res_v.at[j * 8 + i], sem)
                         for i in range(8)]
                for d in descs:
                    d.wait()

            pltpu.sync_copy(res_v, out_ref.at[pl.ds(row0, 88), :])

    return k(sidx2, sval2, gidx2)


def _structure(points):
    N = points.shape[0]
    coords = jnp.floor(points / _VSZ).astype(jnp.int32)
    coords = coords - coords.min(axis=0)
    mx = coords.max(axis=0) + 1
    mx1, mx2 = mx[1], mx[2]
    h = coords[:, 0] * (mx1 * mx2) + coords[:, 1] * mx2 + coords[:, 2]
    uh, inv = jnp.unique(h, return_inverse=True, size=N, fill_value=-1)
    inv = inv.reshape(-1).astype(jnp.int32)
    M = jnp.sum(uh >= 0).astype(jnp.int32)
    row_valid = jnp.arange(N, dtype=jnp.int32) < M
    c0 = uh // (mx1 * mx2)
    r = uh % (mx1 * mx2)
    c1 = r // mx2
    c2 = r % mx2
    vc = jnp.stack([c0, c1, c2], axis=1)
    sidx = jnp.where(row_valid & (uh < _T), uh, _DUMP)
    sval = jnp.arange(N, dtype=jnp.int32)
    sidx2 = jnp.concatenate(
        [sidx, jnp.full((_NSROW * 128 - N,), _DUMP, jnp.int32)]
    ).reshape(_NSROW, 128)
    sval2 = jnp.concatenate(
        [sval, jnp.zeros((_NSROW * 128 - N,), jnp.int32)]
    ).reshape(_NSROW, 128)
    offs = jnp.array(list(_prod((-1, 0, 1), repeat=3)), dtype=jnp.int32)
    nc = vc[None, :, :] + offs[:, None, :]
    valid = (jnp.all((nc >= 0) & (nc < mx[None, None, :]), axis=2)
             & row_valid[None, :])
    nh = nc[..., 0] * (mx1 * mx2) + nc[..., 1] * mx2 + nc[..., 2]
    gq = jnp.where(valid & (nh >= 0) & (nh < _T), nh, _MISS)
    gq2 = jnp.concatenate(
        [gq.reshape(-1),
         jnp.full((_NGROW * 128 - 27 * N,), _MISS, jnp.int32)]
    ).reshape(_NGROW, 128)
    nbr, _ = _sc_table_lookup(sidx2, sval2, gq2)
    neigh = nbr.reshape(-1)[:27 * N].reshape(27, N)
    return inv, M, neigh


def _conv_mm(G, W):
    N = G.shape[1]
    nt = N // _NTILE

    def body(g_ref, w_ref, o_ref):
        k = pl.program_id(1)

        @pl.when(k == 0)
        def _():
            o_ref[...] = jnp.zeros_like(o_ref)

        o_ref[...] += jnp.dot(g_ref[0], w_ref[0],
                              preferred_element_type=jnp.float32)

    return pl.pallas_call(
        body,
        grid=(nt, 27),
        in_specs=[
            pl.BlockSpec((1, _NTILE, _FD), lambda i, k: (k, i, 0)),
            pl.BlockSpec((1, _FD, _FD), lambda i, k: (k, 0, 0)),
        ],
        out_specs=pl.BlockSpec((_NTILE, _FD), lambda i, k: (i, 0)),
        out_shape=jax.ShapeDtypeStruct((N, _FD), jnp.float32),
        compiler_params=pltpu.CompilerParams(
            dimension_semantics=("parallel", "arbitrary")),
    )(G, W)


def kernel(source_points, target_points, enc_W, enc_b, ln_g, ln_b, convW,
           bn_g, bn_b, h1_W, h1_b, h2_W, h2_b):
    sc = source_points - source_points.mean(axis=0, keepdims=True)
    tc = target_points - target_points.mean(axis=0, keepdims=True)
    s_inv, s_M, s_nb = _structure(sc)
    t_inv, t_M, t_nb = _structure(tc)
    Npad = source_points.shape[0]

    def encode(p):
        x = p @ enc_W + enc_b
        m = x.mean(axis=-1, keepdims=True)
        v = ((x - m) ** 2).mean(axis=-1, keepdims=True)
        x = (x - m) / jnp.sqrt(v + 1e-5) * ln_g + ln_b
        return jax.nn.relu(x)

    def vox_mean(feats, inv):
        s = jax.ops.segment_sum(feats, inv, num_segments=Npad)
        c = jax.ops.segment_sum(jnp.ones((feats.shape[0],), feats.dtype),
                                inv, num_segments=Npad)
        return s / jnp.where(c > 0, c, jnp.ones_like(c))[:, None]

    def subm(f, W, nb):
        G = jnp.where(nb[:, :, None] >= 0, f[jnp.clip(nb, 0)], 0.0)
        return _conv_mm(G, W)

    def bn(x, g, b, mask, Mf):
        m = jnp.where(mask[:, None], x, 0.0).sum(axis=0) / Mf
        v = jnp.where(mask[:, None], (x - m) ** 2, 0.0).sum(axis=0) / Mf
        return (x - m) / jnp.sqrt(v + 1e-5) * g + b

    def blocks(f, nb, mask, Mf):
        x = f
        for bi in range(_NBLK):
            idn = x
            y = jax.nn.relu(bn(subm(x, convW[bi, 0], nb),
                               bn_g[bi, 0], bn_b[bi, 0], mask, Mf))
            y = bn(subm(y, convW[bi, 1], nb),
                   bn_g[bi, 1], bn_b[bi, 1], mask, Mf) + idn
            x = jax.nn.relu(y)
        return x

    s_mask = jnp.arange(Npad, dtype=jnp.int32) < s_M
    t_mask = jnp.arange(Npad, dtype=jnp.int32) < t_M
    s_Mf = s_M.astype(jnp.float32)
    t_Mf = t_M.astype(jnp.float32)

    s_feats = blocks(vox_mean(encode(source_points), s_inv), s_nb, s_mask, s_Mf)
    t_feats = blocks(vox_mean(encode(target_points), t_inv), t_nb, t_mask, t_Mf)
    sg = jnp.where(s_mask[:, None], s_feats, -jnp.inf).max(axis=0)
    tg = jnp.where(t_mask[:, None], t_feats, -jnp.inf).max(axis=0)
    comb = sg + tg
    h = jax.nn.relu(comb @ h1_W + h1_b)
    return h @ h2_W + h2_b


# speedup vs baseline: 11.8046x; 1.4920x over previous
import functools

import jax
import jax.numpy as jnp
from jax import lax
from itertools import product as _prod
from jax.experimental import pallas as pl
from jax.experimental.pallas import tpu as pltpu
from jax.experimental.pallas import tpu_sc as plsc

_VSZ = 0.1
_FD = 128
_NBLK = 2
_NTILE = 2000

_UH_PAD = 65536
_BIG = jnp.iinfo(jnp.int32).max
_NGROW = 11264


def _sc_nbsearch(uh_pad, gidx2):
    mesh = plsc.VectorSubcoreMesh(core_axis_name="c", subcore_axis_name="s",
                                  num_cores=2)

    blk = _NGROW * 128 // 32 // 4

    @functools.partial(
        pl.kernel,
        out_type=jax.ShapeDtypeStruct((_NGROW * 128,), jnp.int32),
        mesh=mesh,
        compiler_params=pltpu.CompilerParams(needs_layout_passes=False),
        scratch_types=[
            pltpu.VMEM((_UH_PAD,), jnp.int32),
            pltpu.VMEM((blk,), jnp.int32),
            pltpu.VMEM((blk,), jnp.int32),
        ],
    )
    def k(uh_ref, gidx_ref, out_ref, uh_v, gix_v, res_v):
        w = lax.axis_index("s") * 2 + lax.axis_index("c")
        pltpu.sync_copy(uh_ref, uh_v)

        @pl.loop(0, 4)
        def _(b):
            base = (w * 4 + b) * blk
            pltpu.sync_copy(gidx_ref.at[pl.ds(base, blk)], gix_v)

            @pl.loop(0, blk // 128)
            def _(r):
                qs = [gix_v[pl.ds(r * 128 + j * 16, 16)] for j in range(8)]
                ps = [jnp.zeros((16,), jnp.int32) for _ in range(8)]
                s = _UH_PAD // 2
                while s >= 1:
                    probes = [plsc.load_gather(uh_v, [ps[j] + (s - 1)])
                              for j in range(8)]
                    ps = [jnp.where(probes[j] < qs[j], ps[j] + s, ps[j])
                          for j in range(8)]
                    s //= 2
                for j in range(8):
                    val = plsc.load_gather(uh_v, [ps[j]])
                    res_v[pl.ds(r * 128 + j * 16, 16)] = jnp.where(
                        val == qs[j], ps[j], -1)

            pltpu.sync_copy(res_v, out_ref.at[pl.ds(base, blk)])

    return k(uh_pad, gidx2)


def _structure(points):
    N = points.shape[0]
    coords = jnp.floor(points / _VSZ).astype(jnp.int32)
    coords = coords - coords.min(axis=0)
    mx = coords.max(axis=0) + 1
    mx1, mx2 = mx[1], mx[2]
    h = coords[:, 0] * (mx1 * mx2) + coords[:, 1] * mx2 + coords[:, 2]
    uh, inv = jnp.unique(h, return_inverse=True, size=N, fill_value=-1)
    inv = inv.reshape(-1).astype(jnp.int32)
    M = jnp.sum(uh >= 0).astype(jnp.int32)
    row_valid = jnp.arange(N, dtype=jnp.int32) < M
    c0 = uh // (mx1 * mx2)
    r = uh % (mx1 * mx2)
    c1 = r // mx2
    c2 = r % mx2
    vc = jnp.stack([c0, c1, c2], axis=1)
    uh_s = jnp.where(row_valid, uh, _BIG)
    uh_pad = jnp.concatenate(
        [uh_s, jnp.full((_UH_PAD - N,), _BIG, jnp.int32)])
    offs = jnp.array(list(_prod((-1, 0, 1), repeat=3)), dtype=jnp.int32)
    nc = vc[None, :, :] + offs[:, None, :]
    valid = (jnp.all((nc >= 0) & (nc < mx[None, None, :]), axis=2)
             & row_valid[None, :])
    nh = nc[..., 0] * (mx1 * mx2) + nc[..., 1] * mx2 + nc[..., 2]
    gq = jnp.where(valid, nh, -2)
    gq2 = jnp.concatenate(
        [gq.reshape(-1),
         jnp.full((_NGROW * 128 - 27 * N,), -2, jnp.int32)])
    nbr = _sc_nbsearch(uh_pad, gq2)
    neigh = nbr[:27 * N].reshape(27, N)
    return inv, M, neigh


def _conv_mm(G, W):
    N = G.shape[1]
    nt = N // _NTILE

    def body(g_ref, w_ref, o_ref):
        k = pl.program_id(1)

        @pl.when(k == 0)
        def _():
            o_ref[...] = jnp.zeros_like(o_ref)

        o_ref[...] += jnp.dot(g_ref[0], w_ref[0],
                              preferred_element_type=jnp.float32)

    return pl.pallas_call(
        body,
        grid=(nt, 27),
        in_specs=[
            pl.BlockSpec((1, _NTILE, _FD), lambda i, k: (k, i, 0)),
            pl.BlockSpec((1, _FD, _FD), lambda i, k: (k, 0, 0)),
        ],
        out_specs=pl.BlockSpec((_NTILE, _FD), lambda i, k: (i, 0)),
        out_shape=jax.ShapeDtypeStruct((N, _FD), jnp.float32),
        compiler_params=pltpu.CompilerParams(
            dimension_semantics=("parallel", "arbitrary")),
    )(G, W)


def kernel(source_points, target_points, enc_W, enc_b, ln_g, ln_b, convW,
           bn_g, bn_b, h1_W, h1_b, h2_W, h2_b):
    sc = source_points - source_points.mean(axis=0, keepdims=True)
    tc = target_points - target_points.mean(axis=0, keepdims=True)
    s_inv, s_M, s_nb = _structure(sc)
    t_inv, t_M, t_nb = _structure(tc)
    Npad = source_points.shape[0]

    def encode(p):
        x = p @ enc_W + enc_b
        m = x.mean(axis=-1, keepdims=True)
        v = ((x - m) ** 2).mean(axis=-1, keepdims=True)
        x = (x - m) / jnp.sqrt(v + 1e-5) * ln_g + ln_b
        return jax.nn.relu(x)

    def vox_mean(feats, inv):
        s = jax.ops.segment_sum(feats, inv, num_segments=Npad)
        c = jax.ops.segment_sum(jnp.ones((feats.shape[0],), feats.dtype),
                                inv, num_segments=Npad)
        return s / jnp.where(c > 0, c, jnp.ones_like(c))[:, None]

    def subm(f, W, nb):
        G = jnp.where(nb[:, :, None] >= 0, f[jnp.clip(nb, 0)], 0.0)
        return _conv_mm(G, W)

    def bn(x, g, b, mask, Mf):
        m = jnp.where(mask[:, None], x, 0.0).sum(axis=0) / Mf
        v = jnp.where(mask[:, None], (x - m) ** 2, 0.0).sum(axis=0) / Mf
        return (x - m) / jnp.sqrt(v + 1e-5) * g + b

    def blocks(f, nb, mask, Mf):
        x = f
        for bi in range(_NBLK):
            idn = x
            y = jax.nn.relu(bn(subm(x, convW[bi, 0], nb),
                               bn_g[bi, 0], bn_b[bi, 0], mask, Mf))
            y = bn(subm(y, convW[bi, 1], nb),
                   bn_g[bi, 1], bn_b[bi, 1], mask, Mf) + idn
            x = jax.nn.relu(y)
        return x

    s_mask = jnp.arange(Npad, dtype=jnp.int32) < s_M
    t_mask = jnp.arange(Npad, dtype=jnp.int32) < t_M
    s_Mf = s_M.astype(jnp.float32)
    t_Mf = t_M.astype(jnp.float32)

    s_feats = blocks(vox_mean(encode(source_points), s_inv), s_nb, s_mask, s_Mf)
    t_feats = blocks(vox_mean(encode(target_points), t_inv), t_nb, t_mask, t_Mf)
    sg = jnp.where(s_mask[:, None], s_feats, -jnp.inf).max(axis=0)
    tg = jnp.where(t_mask[:, None], t_feats, -jnp.inf).max(axis=0)
    comb = sg + tg
    h = jax.nn.relu(comb @ h1_W + h1_b)
    return h @ h2_W + h2_b
